# baseline (speedup 1.0000x reference)
import numpy as np
import jax
import jax.numpy as jnp
from jax import lax
from jax.experimental import pallas as pl
from jax.experimental.pallas import tpu as pltpu
from jax.experimental.pallas import tpu_sc as plsc

_N = 800000
_G = 4096
_D2 = 1.0

_LANES = 16
_NW = 32
_VPW = 1562
_CHV = 20
_CHR = _CHV * _LANES
_NFULL = _VPW // _CHV
_REMV = _VPW - _NFULL * _CHV
_TAILR = _N - _NW * _VPW * _LANES
_TAILV = _TAILR // _LANES


def _sc_body(pos_hbm, q_hbm, b_hbm, f_hbm, part_hbm, pb, qb, fb, bb, acc):
    c = lax.axis_index("c")
    s = lax.axis_index("s")
    wid = c * 16 + s

    fzero = jnp.zeros((_LANES,), jnp.float32)

    def zbody(i, carry):
        acc[pl.ds(i * _LANES, _LANES)] = fzero
        return carry

    lax.fori_loop(0, _G // _LANES, zbody, 0)

    i0 = lax.iota(jnp.int32, _LANES)
    c0 = i0 * 0
    c1 = c0 + 1
    c2 = c0 + 2

    def chunk(row0, nvec):
        nrow = nvec * _LANES
        pltpu.sync_copy(pos_hbm.at[pl.ds(row0, nrow)], pb.at[pl.ds(0, nrow)])
        pltpu.sync_copy(q_hbm.at[pl.ds(row0, nrow)], qb.at[pl.ds(0, nrow)])
        pltpu.sync_copy(b_hbm.at[pl.ds(row0, nrow)], bb.at[pl.ds(0, nrow)])

        def vbody(v, carry):
            rows = i0 + v * _LANES
            x = plsc.load_gather(pb, [rows, c0])
            y = plsc.load_gather(pb, [rows, c1])
            z = plsc.load_gather(pb, [rows, c2])
            x0 = plsc.load_gather(qb, [rows, c0])
            y0 = plsc.load_gather(qb, [rows, c1])
            z0 = plsc.load_gather(qb, [rows, c2])
            dx = x - x0
            dy = y - y0
            dz = z - z0
            u = dx * dx - _D2
            plsc.store_scatter(fb, [rows, c0], dx * (u * -4.0))
            plsc.store_scatter(fb, [rows, c1], -dy)
            plsc.store_scatter(fb, [rows, c2], -dz)
            e = u * u + 0.5 * (dy * dy + dz * dz)
            ids = bb[pl.ds(v * _LANES, _LANES)]
            plsc.addupdate_scatter(acc, [ids], e)
            return carry

        lax.fori_loop(0, nvec, vbody, 0)
        pltpu.sync_copy(fb.at[pl.ds(0, nrow)], f_hbm.at[pl.ds(row0, nrow)])

    base = wid * (_VPW * _LANES)

    def cbody(k, carry):
        chunk(base + k * _CHR, _CHV)
        return carry

    lax.fori_loop(0, _NFULL, cbody, 0)
    chunk(base + _NFULL * _CHR, _REMV)

    @pl.when(wid == 0)
    def _tail():
        chunk(jnp.int32(_N - _TAILR), _TAILV)

    pltpu.sync_copy(acc, part_hbm.at[wid])


def _reduce_body(p_ref, o_ref):
    o_ref[...] = jnp.sum(p_ref[...], axis=0, keepdims=True)


def kernel(pos, pos0, batch):
    mesh = plsc.VectorSubcoreMesh(core_axis_name="c", subcore_axis_name="s")
    forces, parts = pl.kernel(
        _sc_body,
        mesh=mesh,
        compiler_params=pltpu.CompilerParams(needs_layout_passes=False),
        out_type=[
            jax.ShapeDtypeStruct((_N, 3), jnp.float32),
            jax.ShapeDtypeStruct((_NW, _G), jnp.float32),
        ],
        scratch_types=[
            pltpu.VMEM((_CHR, 3), jnp.float32),
            pltpu.VMEM((_CHR, 3), jnp.float32),
            pltpu.VMEM((_CHR, 3), jnp.float32),
            pltpu.VMEM((_CHR,), jnp.int32),
            pltpu.VMEM((_G,), jnp.float32),
        ],
    )(pos, pos0, batch)

    energy2d = pl.pallas_call(
        _reduce_body,
        out_shape=jax.ShapeDtypeStruct((1, _G), jnp.float32),
    )(parts)
    return energy2d.reshape(_G), forces
